# baseline (speedup 1.0000x reference)
import functools

import jax
import jax.numpy as jnp
from jax.experimental import pallas as pl
from jax.experimental.pallas import tpu as pltpu

_BS = 4096
_D = 512
_S = 20
_ANO = int(_BS * 0.1)
_BLK = 128
_NSTREAM = 2
_NPAD = 512


def _agg_body(neg_ref, c_ref, n0_ref, n1_ref, w1_ref, w2_ref, o_ref):
    w1 = w1_ref[...].astype(jnp.bfloat16)
    w2 = w2_ref[...].astype(jnp.bfloat16)
    i = pl.program_id(0)
    for k, n_ref in enumerate((n0_ref, n1_ref)):
        c = c_ref[pl.ds(k * _BLK, _BLK), :]
        n = n_ref[...]
        sc_c = jnp.tanh(jnp.dot(c.astype(jnp.bfloat16), w1,
                                preferred_element_type=jnp.float32))
        n2 = n.reshape(_BLK * _S, _D)
        sc_n = jnp.tanh(jnp.dot(n2.astype(jnp.bfloat16), w1,
                                preferred_element_type=jnp.float32))
        weighted = sc_c * c + jnp.sum((sc_n * n2).reshape(_BLK, _S, _D),
                                      axis=1)
        agg = jnp.dot(weighted.astype(jnp.bfloat16), w2,
                      preferred_element_type=jnp.float32)
        base = (i * _NSTREAM + k) * _BLK
        row_ids = base + jax.lax.broadcasted_iota(jnp.int32, (_BLK, _NPAD), 0)
        neg = neg_ref[0, :][None, :]
        is_neg = jnp.any(row_ids == neg, axis=1)
        o_ref[pl.ds(k * _BLK, _BLK), :] = jnp.where(is_neg[:, None], c, agg)


@functools.partial(jax.jit, static_argnums=())
def kernel(center_feat, neighbor_feats, W1, W2):
    bs, d = center_feat.shape
    batch_center = jnp.mean(center_feat, axis=-1)
    diff_center = jnp.sum(center_feat - batch_center[:, None], axis=-1)
    sorted_idx = jnp.argsort(diff_center)
    neg_idx = sorted_idx[bs - _ANO:]

    neg_pad = jnp.full((1, _NPAD), -1, dtype=jnp.int32)
    neg_pad = neg_pad.at[0, : _ANO].set(neg_idx.astype(jnp.int32))

    grid = (bs // (_BLK * _NSTREAM),)
    agg_info = pl.pallas_call(
        _agg_body,
        grid=grid,
        in_specs=[
            pl.BlockSpec((1, _NPAD), lambda i: (0, 0)),
            pl.BlockSpec((_BLK * _NSTREAM, d), lambda i: (i, 0)),
            pl.BlockSpec((_BLK, _S, d), lambda i: (2 * i, 0, 0)),
            pl.BlockSpec((_BLK, _S, d), lambda i: (2 * i + 1, 0, 0)),
            pl.BlockSpec((d, d), lambda i: (0, 0)),
            pl.BlockSpec((d, d), lambda i: (0, 0)),
        ],
        out_specs=pl.BlockSpec((_BLK * _NSTREAM, d), lambda i: (i, 0)),
        out_shape=jax.ShapeDtypeStruct((bs, d), center_feat.dtype),
        compiler_params=pltpu.CompilerParams(
            dimension_semantics=("arbitrary",),
        ),
    )(neg_pad, center_feat, neighbor_feats, neighbor_feats, W1, W2)
    return (agg_info, neg_idx)
